# baseline (speedup 1.0000x reference)
import functools

import jax
import jax.numpy as jnp
from jax import lax
from jax.experimental import pallas as pl
from jax.experimental.pallas import tpu as pltpu
from jax.experimental.pallas import tpu_sc as plsc

N_NODES = 10000
N_EDGES = 320000
DIM = 128

ROW_BLK = 1000

CHUNK = 40
N_WORKERS = 32
EDGES_PER_W = N_EDGES // N_WORKERS
CHUNKS_PER_W = EDGES_PER_W // CHUNK
NSLOT = 5
NROUND = CHUNKS_PER_W // NSLOT
STRIPE_OFF = 624
STRIPE_SZ = 640


def _msg_body(h_ref, w1_ref, b1_ref, w2_ref, b2_ref, o_ref):
    x = h_ref[...]
    m = jnp.dot(x, w1_ref[...], preferred_element_type=jnp.float32)
    m = jnp.maximum(m + b1_ref[...], 0.0)
    m = jnp.dot(m, w2_ref[...], preferred_element_type=jnp.float32)
    o_ref[...] = jnp.maximum(m + b2_ref[...], 0.0)


def _msg_mlp(h, W1, b1, W2, b2):
    full = lambda i: (0, 0)
    return pl.pallas_call(
        _msg_body,
        grid=(N_NODES // ROW_BLK,),
        in_specs=[
            pl.BlockSpec((ROW_BLK, DIM), lambda i: (i, 0)),
            pl.BlockSpec((DIM, DIM), full),
            pl.BlockSpec((1, DIM), full),
            pl.BlockSpec((DIM, DIM), full),
            pl.BlockSpec((1, DIM), full),
        ],
        out_specs=pl.BlockSpec((ROW_BLK, DIM), lambda i: (i, 0)),
        out_shape=jax.ShapeDtypeStruct((N_NODES, DIM), jnp.float32),
    )(h, W1, b1.reshape(1, DIM), W2, b2.reshape(1, DIM))


def _sc_body(msg_hbm, ei_hbm, z128_hbm, ones_hbm,
             agg_out, deg_out,
             cb0, cb1, cb2, cb3, cb4, rb0, rb1, rb2, rb3, rb4,
             b0, b1, b2, b3, b4, acc_sh,
             i0, i1, i2, i3, i4, g0, g1, g2, g3, g4, a0, a1, a2, a3, a4):
    cbs = [cb0, cb1, cb2, cb3, cb4]
    rbs = [rb0, rb1, rb2, rb3, rb4]
    bufs = [b0, b1, b2, b3, b4]
    isem = [i0, i1, i2, i3, i4]
    gsem = [g0, g1, g2, g3, g4]
    asem = [a0, a1, a2, a3, a4]

    c = lax.axis_index("c")
    s = lax.axis_index("s")
    wid = c * 16 + s
    base = wid * EDGES_PER_W
    r0 = s * STRIPE_OFF
    out0 = c * N_NODES + r0

    def start_idx(j, b, with_col):
        off = base + j * CHUNK
        if with_col:
            pltpu.async_copy(ei_hbm.at[pl.ds(N_EDGES + off, CHUNK)], cbs[b], isem[b])
        pltpu.async_copy(ei_hbm.at[pl.ds(off, CHUNK)], rbs[b], isem[b])

    def wait_idx(j, b, with_col):
        off = base + j * CHUNK
        if with_col:
            pltpu.make_async_copy(ei_hbm.at[pl.ds(N_EDGES + off, CHUNK)], cbs[b], isem[b]).wait()
        pltpu.make_async_copy(ei_hbm.at[pl.ds(off, CHUNK)], rbs[b], isem[b]).wait()

    def start_gather(b):
        pltpu.async_copy(msg_hbm.at[cbs[b]], bufs[b], gsem[b])

    def wait_gather(b):
        pltpu.make_async_copy(msg_hbm.at[cbs[b]], bufs[b], gsem[b]).wait()

    def start_scat(b, src):
        pltpu.async_copy(src, acc_sh.at[rbs[b]], asem[b], add=True)

    def wait_scat(b, src):
        pltpu.make_async_copy(src, acc_sh.at[rbs[b]], asem[b]).wait()

    pltpu.sync_copy(z128_hbm.at[pl.ds(r0, STRIPE_SZ)], acc_sh.at[pl.ds(r0, STRIPE_SZ)])
    plsc.subcore_barrier()

    start_idx(0, 0, True)
    start_idx(1, 1, True)
    start_idx(2, 2, True)
    wait_idx(0, 0, True)
    start_gather(0)
    wait_idx(1, 1, True)
    start_gather(1)

    def body_a(t, carry):
        for b in range(NSLOT):
            j = t * NSLOT + b
            bi = (b + 3) % NSLOT
            bg = (b + 2) % NSLOT
            if b < 2:
                @pl.when(t > 0)
                def _wa():
                    wait_scat(bi, bufs[bi])
            else:
                wait_scat(bi, bufs[bi])
            if b < 2:
                start_idx(j + 3, bi, True)
            else:
                @pl.when(t < NROUND - 1)
                def _si():
                    start_idx(j + 3, bi, True)
            if b < 3:
                wait_idx(j + 2, bg, True)
                start_gather(bg)
            else:
                @pl.when(t < NROUND - 1)
                def _sg():
                    wait_idx(j + 2, bg, True)
                    start_gather(bg)
            wait_gather(b)
            start_scat(b, bufs[b])
        return carry

    lax.fori_loop(0, NROUND, body_a, 0)
    wait_scat(3, bufs[3])
    wait_scat(4, bufs[4])
    plsc.subcore_barrier()
    pltpu.sync_copy(acc_sh.at[pl.ds(r0, STRIPE_SZ)], agg_out.at[pl.ds(out0, STRIPE_SZ)])
    plsc.subcore_barrier()

    pltpu.sync_copy(z128_hbm.at[pl.ds(r0, STRIPE_SZ)], acc_sh.at[pl.ds(r0, STRIPE_SZ)])
    pltpu.sync_copy(ones_hbm, bufs[0])
    plsc.subcore_barrier()

    start_idx(0, 0, False)
    start_idx(1, 1, False)
    start_idx(2, 2, False)

    def body_b(t, carry):
        for b in range(NSLOT):
            j = t * NSLOT + b
            bi = (b + 3) % NSLOT
            if b < 2:
                @pl.when(t > 0)
                def _wb():
                    wait_scat(bi, bufs[0])
            else:
                wait_scat(bi, bufs[0])
            if b < 2:
                start_idx(j + 3, bi, False)
            else:
                @pl.when(t < NROUND - 1)
                def _sj():
                    start_idx(j + 3, bi, False)
            wait_idx(j, b, False)
            start_scat(b, bufs[0])
        return carry

    lax.fori_loop(0, NROUND, body_b, 0)
    wait_scat(3, bufs[0])
    wait_scat(4, bufs[0])
    plsc.subcore_barrier()
    pltpu.sync_copy(acc_sh.at[pl.ds(r0, STRIPE_SZ)], deg_out.at[pl.ds(out0, STRIPE_SZ)])


def _sc_aggregate(msg, ei_flat):
    mesh = plsc.VectorSubcoreMesh(core_axis_name="c", subcore_axis_name="s")
    z128 = jnp.zeros((N_NODES, DIM), jnp.float32)
    ones = jnp.ones((CHUNK, DIM), jnp.float32)
    k = functools.partial(
        pl.kernel,
        mesh=mesh,
        out_type=[
            jax.ShapeDtypeStruct((2 * N_NODES, DIM), jnp.float32),
            jax.ShapeDtypeStruct((2 * N_NODES, DIM), jnp.float32),
        ],
        scratch_types=(
            [pltpu.VMEM((CHUNK,), jnp.int32) for _ in range(2 * NSLOT)]
            + [pltpu.VMEM((CHUNK, DIM), jnp.float32) for _ in range(NSLOT)]
            + [pltpu.VMEM_SHARED((N_NODES, DIM), jnp.float32)]
            + [pltpu.SemaphoreType.DMA for _ in range(3 * NSLOT)]
        ),
    )(_sc_body)
    return k(msg, ei_flat, z128, ones)


def _upd_body(h_ref, a0_ref, a1_ref, d0_ref, d1_ref,
              w3h_ref, w3a_ref, b3_ref, w4_ref, b4_ref, o_ref):
    agg = a0_ref[...] + a1_ref[...]
    deg = d0_ref[:, 0:1] + d1_ref[:, 0:1]
    deg = jnp.maximum(deg, 1.0)
    agg = agg / deg
    z = jnp.dot(h_ref[...], w3h_ref[...], preferred_element_type=jnp.float32)
    z = z + jnp.dot(agg, w3a_ref[...], preferred_element_type=jnp.float32)
    z = jnp.maximum(z + b3_ref[...], 0.0)
    out = jnp.dot(z, w4_ref[...], preferred_element_type=jnp.float32)
    o_ref[...] = out + b4_ref[...]


def _update_mlp(h, agg2, deg2, W3, b3, W4, b4):
    full = lambda i: (0, 0)
    blk = lambda i: (i, 0)
    hi = lambda i: (i + N_NODES // ROW_BLK, 0)
    return pl.pallas_call(
        _upd_body,
        grid=(N_NODES // ROW_BLK,),
        in_specs=[
            pl.BlockSpec((ROW_BLK, DIM), blk),
            pl.BlockSpec((ROW_BLK, DIM), blk),
            pl.BlockSpec((ROW_BLK, DIM), hi),
            pl.BlockSpec((ROW_BLK, DIM), blk),
            pl.BlockSpec((ROW_BLK, DIM), hi),
            pl.BlockSpec((DIM, DIM), full),
            pl.BlockSpec((DIM, DIM), full),
            pl.BlockSpec((1, DIM), full),
            pl.BlockSpec((DIM, DIM), full),
            pl.BlockSpec((1, DIM), full),
        ],
        out_specs=pl.BlockSpec((ROW_BLK, DIM), blk),
        out_shape=jax.ShapeDtypeStruct((N_NODES, DIM), jnp.float32),
    )(h, agg2, agg2, deg2, deg2, W3[:DIM], W3[DIM:],
      b3.reshape(1, DIM), W4, b4.reshape(1, DIM))


def kernel(h, edge_index, W1, b1, W2, b2, W3, b3, W4, b4):
    ei_flat = edge_index.astype(jnp.int32).reshape(2 * N_EDGES)
    msg = _msg_mlp(h, W1, b1, W2, b2)
    agg2, deg2 = _sc_aggregate(msg, ei_flat)
    return _update_mlp(h, agg2, deg2, W3, b3, W4, b4)


# speedup vs baseline: 1.0568x; 1.0568x over previous
import functools

import jax
import jax.numpy as jnp
from jax import lax
from jax.experimental import pallas as pl
from jax.experimental.pallas import tpu as pltpu
from jax.experimental.pallas import tpu_sc as plsc

N_NODES = 10000
N_EDGES = 320000
DIM = 128

ROW_BLK = 1000

CHUNK = 80
N_WORKERS = 32
EDGES_PER_W = N_EDGES // N_WORKERS
CHUNKS_PER_W = EDGES_PER_W // CHUNK
NSLOT = 4
NROUND = (CHUNKS_PER_W - 1) // NSLOT
STRIPE_OFF = 624
STRIPE_SZ = 640


def _msg_body(h_ref, w1_ref, b1_ref, w2_ref, b2_ref, o_ref):
    x = h_ref[...]
    m = jnp.dot(x, w1_ref[...], preferred_element_type=jnp.float32)
    m = jnp.maximum(m + b1_ref[...], 0.0)
    m = jnp.dot(m, w2_ref[...], preferred_element_type=jnp.float32)
    o_ref[...] = jnp.maximum(m + b2_ref[...], 0.0)


def _msg_mlp(h, W1, b1, W2, b2):
    full = lambda i: (0, 0)
    return pl.pallas_call(
        _msg_body,
        grid=(N_NODES // ROW_BLK,),
        in_specs=[
            pl.BlockSpec((ROW_BLK, DIM), lambda i: (i, 0)),
            pl.BlockSpec((DIM, DIM), full),
            pl.BlockSpec((1, DIM), full),
            pl.BlockSpec((DIM, DIM), full),
            pl.BlockSpec((1, DIM), full),
        ],
        out_specs=pl.BlockSpec((ROW_BLK, DIM), lambda i: (i, 0)),
        out_shape=jax.ShapeDtypeStruct((N_NODES, DIM), jnp.float32),
    )(h, W1, b1.reshape(1, DIM), W2, b2.reshape(1, DIM))


def _sc_body(msg_hbm, ei_hbm, z128_hbm, ones_hbm,
             agg_out, deg_out,
             cb0, cb1, cb2, cb3, rb0, rb1, rb2, rb3,
             b0, b1, b2, b3, acc_sh,
             i0, i1, i2, i3, g0, g1, g2, g3, a0, a1, a2, a3):
    cbs = [cb0, cb1, cb2, cb3]
    rbs = [rb0, rb1, rb2, rb3]
    bufs = [b0, b1, b2, b3]
    isem = [i0, i1, i2, i3]
    gsem = [g0, g1, g2, g3]
    asem = [a0, a1, a2, a3]

    c = lax.axis_index("c")
    s = lax.axis_index("s")
    wid = c * 16 + s
    base = wid * EDGES_PER_W
    r0 = s * STRIPE_OFF
    out0 = c * N_NODES + r0

    def start_idx(j, b, with_col):
        off = base + j * CHUNK
        if with_col:
            pltpu.async_copy(ei_hbm.at[pl.ds(N_EDGES + off, CHUNK)], cbs[b], isem[b])
        pltpu.async_copy(ei_hbm.at[pl.ds(off, CHUNK)], rbs[b], isem[b])

    def wait_idx(j, b, with_col):
        off = base + j * CHUNK
        if with_col:
            pltpu.make_async_copy(ei_hbm.at[pl.ds(N_EDGES + off, CHUNK)], cbs[b], isem[b]).wait()
        pltpu.make_async_copy(ei_hbm.at[pl.ds(off, CHUNK)], rbs[b], isem[b]).wait()

    def start_gather(b):
        pltpu.async_copy(msg_hbm.at[cbs[b]], bufs[b], gsem[b])

    def wait_gather(b):
        pltpu.make_async_copy(msg_hbm.at[cbs[b]], bufs[b], gsem[b]).wait()

    def start_scat(b, src):
        pltpu.async_copy(src, acc_sh.at[rbs[b]], asem[b], add=True)

    def wait_scat(b, src):
        pltpu.make_async_copy(src, acc_sh.at[rbs[b]], asem[b]).wait()

    pltpu.sync_copy(z128_hbm.at[pl.ds(r0, STRIPE_SZ)], acc_sh.at[pl.ds(r0, STRIPE_SZ)])
    plsc.subcore_barrier()

    start_idx(0, 0, True)
    start_idx(1, 1, True)
    wait_idx(0, 0, True)
    start_gather(0)

    def body_a(t, carry):
        for b in range(NSLOT):
            j = t * NSLOT + b
            bi = (b + 2) % NSLOT
            bg = (b + 1) % NSLOT
            if b < 2:
                @pl.when(t > 0)
                def _wa():
                    wait_scat(bi, bufs[bi])
            else:
                wait_scat(bi, bufs[bi])
            if b == 3:
                @pl.when(t < NROUND - 1)
                def _si():
                    start_idx(j + 2, bi, True)
            else:
                start_idx(j + 2, bi, True)
            wait_idx(j + 1, bg, True)
            start_gather(bg)
            wait_gather(b)
            start_scat(b, bufs[b])
        return carry

    lax.fori_loop(0, NROUND, body_a, 0)
    wait_gather(0)
    start_scat(0, bufs[0])
    wait_scat(2, bufs[2])
    wait_scat(3, bufs[3])
    wait_scat(0, bufs[0])
    plsc.subcore_barrier()
    pltpu.sync_copy(acc_sh.at[pl.ds(r0, STRIPE_SZ)], agg_out.at[pl.ds(out0, STRIPE_SZ)])
    plsc.subcore_barrier()

    pltpu.sync_copy(z128_hbm.at[pl.ds(r0, STRIPE_SZ)], acc_sh.at[pl.ds(r0, STRIPE_SZ)])
    pltpu.sync_copy(ones_hbm, bufs[0])
    plsc.subcore_barrier()

    start_idx(0, 0, False)
    start_idx(1, 1, False)

    def body_b(t, carry):
        for b in range(NSLOT):
            j = t * NSLOT + b
            bi = (b + 2) % NSLOT
            if b < 2:
                @pl.when(t > 0)
                def _wb():
                    wait_scat(bi, bufs[0])
            else:
                wait_scat(bi, bufs[0])
            if b == 3:
                @pl.when(t < NROUND - 1)
                def _sj():
                    start_idx(j + 2, bi, False)
            else:
                start_idx(j + 2, bi, False)
            wait_idx(j, b, False)
            start_scat(b, bufs[0])
        return carry

    lax.fori_loop(0, NROUND, body_b, 0)
    wait_idx(CHUNKS_PER_W - 1, 0, False)
    start_scat(0, bufs[0])
    wait_scat(2, bufs[0])
    wait_scat(3, bufs[0])
    wait_scat(0, bufs[0])
    plsc.subcore_barrier()
    pltpu.sync_copy(acc_sh.at[pl.ds(r0, STRIPE_SZ)], deg_out.at[pl.ds(out0, STRIPE_SZ)])


def _sc_aggregate(msg, ei_flat):
    mesh = plsc.VectorSubcoreMesh(core_axis_name="c", subcore_axis_name="s")
    z128 = jnp.zeros((N_NODES, DIM), jnp.float32)
    ones = jnp.ones((CHUNK, DIM), jnp.float32)
    k = functools.partial(
        pl.kernel,
        mesh=mesh,
        out_type=[
            jax.ShapeDtypeStruct((2 * N_NODES, DIM), jnp.float32),
            jax.ShapeDtypeStruct((2 * N_NODES, DIM), jnp.float32),
        ],
        scratch_types=(
            [pltpu.VMEM((CHUNK,), jnp.int32) for _ in range(2 * NSLOT)]
            + [pltpu.VMEM((CHUNK, DIM), jnp.float32) for _ in range(NSLOT)]
            + [pltpu.VMEM_SHARED((N_NODES, DIM), jnp.float32)]
            + [pltpu.SemaphoreType.DMA for _ in range(3 * NSLOT)]
        ),
    )(_sc_body)
    return k(msg, ei_flat, z128, ones)


def _upd_body(h_ref, a0_ref, a1_ref, d0_ref, d1_ref,
              w3h_ref, w3a_ref, b3_ref, w4_ref, b4_ref, o_ref):
    agg = a0_ref[...] + a1_ref[...]
    deg = d0_ref[:, 0:1] + d1_ref[:, 0:1]
    deg = jnp.maximum(deg, 1.0)
    agg = agg / deg
    z = jnp.dot(h_ref[...], w3h_ref[...], preferred_element_type=jnp.float32)
    z = z + jnp.dot(agg, w3a_ref[...], preferred_element_type=jnp.float32)
    z = jnp.maximum(z + b3_ref[...], 0.0)
    out = jnp.dot(z, w4_ref[...], preferred_element_type=jnp.float32)
    o_ref[...] = out + b4_ref[...]


def _update_mlp(h, agg2, deg2, W3, b3, W4, b4):
    full = lambda i: (0, 0)
    blk = lambda i: (i, 0)
    hi = lambda i: (i + N_NODES // ROW_BLK, 0)
    return pl.pallas_call(
        _upd_body,
        grid=(N_NODES // ROW_BLK,),
        in_specs=[
            pl.BlockSpec((ROW_BLK, DIM), blk),
            pl.BlockSpec((ROW_BLK, DIM), blk),
            pl.BlockSpec((ROW_BLK, DIM), hi),
            pl.BlockSpec((ROW_BLK, DIM), blk),
            pl.BlockSpec((ROW_BLK, DIM), hi),
            pl.BlockSpec((DIM, DIM), full),
            pl.BlockSpec((DIM, DIM), full),
            pl.BlockSpec((1, DIM), full),
            pl.BlockSpec((DIM, DIM), full),
            pl.BlockSpec((1, DIM), full),
        ],
        out_specs=pl.BlockSpec((ROW_BLK, DIM), blk),
        out_shape=jax.ShapeDtypeStruct((N_NODES, DIM), jnp.float32),
    )(h, agg2, agg2, deg2, deg2, W3[:DIM], W3[DIM:],
      b3.reshape(1, DIM), W4, b4.reshape(1, DIM))


def kernel(h, edge_index, W1, b1, W2, b2, W3, b3, W4, b4):
    ei_flat = edge_index.astype(jnp.int32).reshape(2 * N_EDGES)
    msg = _msg_mlp(h, W1, b1, W2, b2)
    agg2, deg2 = _sc_aggregate(msg, ei_flat)
    return _update_mlp(h, agg2, deg2, W3, b3, W4, b4)
